# baseline (speedup 1.0000x reference)
import functools

import jax
import jax.numpy as jnp
from jax import lax
from jax.experimental import pallas as pl
from jax.experimental.pallas import tpu as pltpu
import jax.experimental.pallas.tpu_sc as plsc

_N = 4096
_K = 8192
_D = 32
_ROWS = 512
_COLS = 1024
_NCH = _K // _COLS
_KLD_SCALE = 10.0


def _argmin_body(beta_ref, x_ref, cb_ref, near_ref, loss_ref, perp_ref,
                 counts_ref, cn_ref, acc_ref):
    i = pl.program_id(0)
    xb = x_ref[0]
    xn = jnp.sum(xb * xb, axis=0, keepdims=True)

    @pl.when(i == 0)
    def _():
        counts_ref[...] = jnp.zeros_like(counts_ref)
        acc_ref[0] = 0.0
        for c in range(_NCH):
            cb = cb_ref[pl.ds(c * _COLS, _COLS), :]
            cn_ref[:, pl.ds(c, 1)] = jnp.sum(cb * cb, axis=1, keepdims=True)

    best_d = jnp.full((1, _ROWS), jnp.inf, dtype=jnp.float32)
    best_i = jnp.zeros((1, _ROWS), dtype=jnp.int32)
    for c in range(_NCH):
        cb = cb_ref[pl.ds(c * _COLS, _COLS), :]
        cn = cn_ref[:, pl.ds(c, 1)]
        m = lax.dot_general(cb, xb, (((1,), (0,)), ((), ())),
                            preferred_element_type=jnp.float32)
        d = (xn + cn) - 2.0 * m
        dmin = jnp.min(d, axis=0, keepdims=True)
        iota = lax.broadcasted_iota(jnp.int32, (_COLS, _ROWS), 0)
        idx = jnp.min(jnp.where(d == dmin, iota, _K), axis=0, keepdims=True)
        take = dmin < best_d
        best_d = jnp.where(take, dmin, best_d)
        best_i = jnp.where(take, idx + c * _COLS, best_i)

    near_ref[...] = best_i.reshape(1, 1, _ROWS)

    low = best_i & (_COLS - 1)
    high = best_i >> 10
    iota0 = lax.broadcasted_iota(jnp.int32, (_COLS, _ROWS), 0)
    oh_low = (low == iota0).astype(jnp.float32)
    iotac = lax.broadcasted_iota(jnp.int32, (_NCH, _ROWS), 0)
    oh_high = (high == iotac).astype(jnp.float32)
    counts_ref[...] += lax.dot_general(
        oh_low, oh_high, (((1,), (1,)), ((), ())),
        preferred_element_type=jnp.float32)

    acc_ref[0] += jnp.sum(best_d)

    @pl.when(i == pl.num_programs(0) - 1)
    def _():
        mse = acc_ref[0] / jnp.float32(_N * _D)
        beta = beta_ref[0]
        loss_ref[0, 0] = (beta * mse + mse) * jnp.float32(_KLD_SCALE)
        e_mean = counts_ref[...] / jnp.float32(_N)
        ent = e_mean * jnp.log(e_mean + 1e-10)
        perp_ref[0, 0] = jnp.exp(-jnp.sum(ent))


def _nearest_and_stats(x3d, codebook, beta):
    grid = _N // _ROWS
    hw_ch = 1024 // _ROWS
    near, loss, perp = pl.pallas_call(
        _argmin_body,
        grid=(grid,),
        in_specs=[
            pl.BlockSpec(memory_space=pltpu.SMEM),
            pl.BlockSpec((1, _D, _ROWS), lambda i: (i // hw_ch, 0, i % hw_ch)),
            pl.BlockSpec((_K, _D), lambda i: (0, 0)),
        ],
        out_specs=[
            pl.BlockSpec((1, 1, _ROWS), lambda i: (i, 0, 0)),
            pl.BlockSpec(memory_space=pltpu.SMEM),
            pl.BlockSpec(memory_space=pltpu.SMEM),
        ],
        out_shape=[
            jax.ShapeDtypeStruct((grid, 1, _ROWS), jnp.int32),
            jax.ShapeDtypeStruct((1, 1), jnp.float32),
            jax.ShapeDtypeStruct((1, 1), jnp.float32),
        ],
        scratch_shapes=[
            pltpu.VMEM((_COLS, _NCH), jnp.float32),
            pltpu.VMEM((_COLS, _NCH), jnp.float32),
            pltpu.SMEM((1,), jnp.float32),
        ],
    )(beta, x3d, codebook)
    return near.reshape(_N), loss[0, 0], perp[0, 0]


def _make_sc_gather():
    info = plsc.get_sparse_core_info()
    nw = info.num_cores * info.num_subcores
    rpw = _N // nw
    mesh = plsc.VectorSubcoreMesh(core_axis_name="c", subcore_axis_name="s")

    @functools.partial(
        pl.kernel,
        mesh=mesh,
        out_type=jax.ShapeDtypeStruct((4, _D, 1024), jnp.float32),
        scratch_types=[
            pltpu.VMEM((rpw,), jnp.int32),
            pltpu.VMEM((rpw, _D), jnp.float32),
            pltpu.VMEM((_D, rpw), jnp.float32),
            pltpu.SemaphoreType.DMA,
        ],
        compiler_params=pltpu.CompilerParams(use_tc_tiling_on_sc=False,
                                             needs_layout_passes=False),
    )
    def gather(cb_hbm, idx_hbm, out_hbm, idx_v, rows_v, qt_v, sem):
        wid = lax.axis_index("s") * info.num_cores + lax.axis_index("c")
        base = wid * rpw
        b = base // 1024
        hw0 = base % 1024
        pltpu.sync_copy(idx_hbm.at[pl.ds(base, rpw)], idx_v)
        pltpu.async_copy(cb_hbm.at[idx_v], rows_v, sem).wait()
        lane = lax.iota(jnp.int32, 16)
        for c in range(_D):
            for k in range(rpw // 16):
                v = plsc.load_gather(rows_v, [lane + k * 16,
                                              jnp.full((16,), c, jnp.int32)])
                qt_v[c, pl.ds(k * 16, 16)] = v
        pltpu.sync_copy(qt_v, out_hbm.at[b, :, pl.ds(hw0, rpw)])

    return gather


def kernel(inputs, beta, codebook):
    x3d = inputs.reshape(4, _D, 1024)
    beta1 = beta.reshape(1)

    near, loss, perp = _nearest_and_stats(x3d, codebook, beta1)
    q_out = _make_sc_gather()(codebook, near).reshape(inputs.shape)
    return (loss, q_out, perp)


# speedup vs baseline: 1.0316x; 1.0316x over previous
import functools

import jax
import jax.numpy as jnp
from jax import lax
from jax.experimental import pallas as pl
from jax.experimental.pallas import tpu as pltpu
import jax.experimental.pallas.tpu_sc as plsc

_N = 4096
_K = 8192
_D = 32
_ROWS = 512
_COLS = 1024
_NCH = _K // _COLS
_KLD_SCALE = 10.0


def _argmin_body(beta_ref, x_ref, cb_ref, near_ref, loss_ref, perp_ref,
                 counts_ref, cn_ref, acc_ref):
    i = pl.program_id(0)
    xb = x_ref[0]
    xn = jnp.sum(xb * xb, axis=0, keepdims=True)

    @pl.when(i == 0)
    def _():
        counts_ref[...] = jnp.zeros_like(counts_ref)
        acc_ref[0] = 0.0
        for c in range(_NCH):
            cb = cb_ref[pl.ds(c * _COLS, _COLS), :]
            cn_ref[:, pl.ds(c, 1)] = jnp.sum(cb * cb, axis=1, keepdims=True)

    best_d = jnp.full((1, _ROWS), jnp.inf, dtype=jnp.float32)
    best_i = jnp.zeros((1, _ROWS), dtype=jnp.int32)
    for c in range(_NCH):
        cb = cb_ref[pl.ds(c * _COLS, _COLS), :]
        cn = cn_ref[:, pl.ds(c, 1)]
        m = lax.dot_general(cb, xb, (((1,), (0,)), ((), ())),
                            preferred_element_type=jnp.float32)
        d = (xn + cn) - 2.0 * m
        dmin = jnp.min(d, axis=0, keepdims=True)
        iota = lax.broadcasted_iota(jnp.int32, (_COLS, _ROWS), 0)
        idx = jnp.min(jnp.where(d == dmin, iota, _K), axis=0, keepdims=True)
        take = dmin < best_d
        best_d = jnp.where(take, dmin, best_d)
        best_i = jnp.where(take, idx + c * _COLS, best_i)

    near_ref[...] = best_i.reshape(1, 1, _ROWS)

    low = best_i & (_COLS - 1)
    high = best_i >> 10
    iota0 = lax.broadcasted_iota(jnp.int32, (_COLS, _ROWS), 0)
    oh_low = (low == iota0).astype(jnp.float32)
    iotac = lax.broadcasted_iota(jnp.int32, (_NCH, _ROWS), 0)
    oh_high = (high == iotac).astype(jnp.float32)
    counts_ref[...] += lax.dot_general(
        oh_low, oh_high, (((1,), (1,)), ((), ())),
        preferred_element_type=jnp.float32)

    acc_ref[0] += jnp.sum(best_d)

    @pl.when(i == pl.num_programs(0) - 1)
    def _():
        mse = acc_ref[0] / jnp.float32(_N * _D)
        beta = beta_ref[0]
        loss_ref[0, 0] = (beta * mse + mse) * jnp.float32(_KLD_SCALE)
        e_mean = counts_ref[...] / jnp.float32(_N)
        ent = e_mean * jnp.log(e_mean + 1e-10)
        perp_ref[0, 0] = jnp.exp(-jnp.sum(ent))


def _nearest_and_stats(x3d, codebook, beta):
    grid = _N // _ROWS
    hw_ch = 1024 // _ROWS
    near, loss, perp = pl.pallas_call(
        _argmin_body,
        grid=(grid,),
        in_specs=[
            pl.BlockSpec(memory_space=pltpu.SMEM),
            pl.BlockSpec((1, _D, _ROWS), lambda i: (i // hw_ch, 0, i % hw_ch)),
            pl.BlockSpec((_K, _D), lambda i: (0, 0)),
        ],
        out_specs=[
            pl.BlockSpec((1, 1, _ROWS), lambda i: (i, 0, 0)),
            pl.BlockSpec(memory_space=pltpu.SMEM),
            pl.BlockSpec(memory_space=pltpu.SMEM),
        ],
        out_shape=[
            jax.ShapeDtypeStruct((grid, 1, _ROWS), jnp.int32),
            jax.ShapeDtypeStruct((1, 1), jnp.float32),
            jax.ShapeDtypeStruct((1, 1), jnp.float32),
        ],
        scratch_shapes=[
            pltpu.VMEM((_COLS, _NCH), jnp.float32),
            pltpu.VMEM((_COLS, _NCH), jnp.float32),
            pltpu.SMEM((1,), jnp.float32),
        ],
    )(beta, x3d, codebook)
    return near.reshape(_N), loss[0, 0], perp[0, 0]


def _make_sc_gather():
    info = plsc.get_sparse_core_info()
    nw = info.num_cores * info.num_subcores
    rpw = _N // nw
    mesh = plsc.VectorSubcoreMesh(core_axis_name="c", subcore_axis_name="s")

    @functools.partial(
        pl.kernel,
        mesh=mesh,
        out_type=jax.ShapeDtypeStruct((_N, _D), jnp.float32),
        scratch_types=[
            pltpu.VMEM((rpw,), jnp.int32),
            pltpu.VMEM((rpw, _D), jnp.float32),
            pltpu.SemaphoreType.DMA,
        ],
        compiler_params=pltpu.CompilerParams(use_tc_tiling_on_sc=False),
    )
    def gather(cb_hbm, idx_hbm, out_hbm, idx_v, rows_v, sem):
        wid = lax.axis_index("s") * info.num_cores + lax.axis_index("c")
        base = wid * rpw
        pltpu.sync_copy(idx_hbm.at[pl.ds(base, rpw)], idx_v)
        pltpu.async_copy(cb_hbm.at[idx_v], rows_v, sem).wait()
        pltpu.sync_copy(rows_v, out_hbm.at[pl.ds(base, rpw)])

    return gather


def kernel(inputs, beta, codebook):
    x3d = inputs.reshape(4, _D, 1024)
    beta1 = beta.reshape(1)

    near, loss, perp = _nearest_and_stats(x3d, codebook, beta1)
    quant2d = _make_sc_gather()(codebook, near)
    q_out = jnp.transpose(quant2d.reshape(4, 1024, _D), (0, 2, 1)).reshape(
        inputs.shape)
    return (loss, q_out, perp)
